# baseline (speedup 1.0000x reference)
import jax
import jax.numpy as jnp
from jax import lax
from jax.experimental import pallas as pl
from jax.experimental.pallas import tpu as pltpu
from jax.experimental.pallas import tpu_sc as plsc

N = 10000
E = 320000
F = 128
B = 10000
BP = 10240
NC, NS = 2, 16
C = 128
NCHUNK = E // C
PAIRS = 78


def _sc_body(nodes, edges_all, feat_cat,
             out_self, partial,
             sum_sp, deg_sp, rowsA, rowsB, idxA, idxB,
             dstA, dstB, nidx, winv, dbuf, ones_v,
             sIA, sIB, sGA, sGB, sOA, sOB, sSA, sSB):
    c = lax.axis_index("c")
    s = lax.axis_index("s")
    coff = c * N
    zero16 = jnp.zeros((16,), jnp.float32)
    one16 = jnp.ones((16,), jnp.float32)

    for k in range(8):
        ones_v[pl.ds(k * 16, 16)] = one16
    for j in range(5):
        pltpu.sync_copy(nodes.at[pl.ds(s * 640 + j * C, C)], nidx.at[j])

    def issue_idx(g, ch, idxbuf, sem):
        ch = jnp.minimum(ch, NCHUNK - 1)
        pltpu.async_copy(edges_all.at[g, :, pl.ds(ch * C, C)], idxbuf, sem)

    def wait_idx(idxbuf, sem):
        pltpu.make_async_copy(
            edges_all.at[0, :, pl.ds(0, C)], idxbuf, sem).wait()

    def stage1(idxbuf, dstbuf, rowsbuf, semO, semG):
        for k in range(8):
            dstbuf[pl.ds(k * 16, 16)] = idxbuf[1, pl.ds(k * 16, 16)]
        for k in range(8):
            idxbuf[0, pl.ds(k * 16, 16)] = idxbuf[0, pl.ds(k * 16, 16)] + coff
        pltpu.async_copy(ones_v, deg_sp.at[dstbuf], semO, add=True)
        pltpu.async_copy(feat_cat.at[idxbuf.at[0]], rowsbuf, semG)

    def wait_g(idxbuf, rowsbuf, sem):
        pltpu.make_async_copy(feat_cat.at[idxbuf.at[0]], rowsbuf, sem).wait()

    def issue_sc(rowsbuf, dstbuf, sem):
        pltpu.async_copy(rowsbuf, sum_sp.at[dstbuf], sem, add=True)

    def wait_sc(rowsbuf, dstbuf, sem):
        pltpu.make_async_copy(rowsbuf, sum_sp.at[dstbuf], sem).wait()

    def wait_ones(dstbuf, sem):
        pltpu.make_async_copy(ones_v, deg_sp.at[dstbuf], sem).wait()

    lo = s * NCHUNK // NS
    nfull = jnp.minimum(640, N - s * 640) // C
    has_rem = s % 4 == 3

    def graph_body(g, carry):
        def zw(k, cr):
            winv[pl.ds(k * 16, 16)] = zero16
            return cr
        lax.fori_loop(0, 40, zw, 0)
        pltpu.sync_copy(winv.at[pl.ds(0, 640)], deg_sp.at[pl.ds(s * 640, 640)])

        def zr(r, cr):
            for j2 in range(8):
                rowsA[r, pl.ds(j2 * 16, 16)] = zero16
            return cr
        lax.fori_loop(0, C, zr, 0)

        def zacc(i, cr):
            pltpu.sync_copy(rowsA, sum_sp.at[pl.ds(s * 640 + i * C, C)])
            return cr
        lax.fori_loop(0, nfull, zacc, 0)

        @pl.when(s == NS - 1)
        def _():
            pltpu.sync_copy(rowsA.at[pl.ds(0, 16)], sum_sp.at[pl.ds(9984, 16)])
        plsc.subcore_barrier()

        issue_idx(g, lo, idxA, sIA)
        issue_idx(g, lo + 1, idxB, sIB)
        wait_idx(idxA, sIA)
        stage1(idxA, dstA, rowsA, sOA, sGA)
        wait_idx(idxB, sIB)
        stage1(idxB, dstB, rowsB, sOB, sGB)
        wait_g(idxA, rowsA, sGA)
        issue_sc(rowsA, dstA, sSA)
        issue_idx(g, lo + 2, idxA, sIA)
        wait_g(idxB, rowsB, sGB)
        issue_sc(rowsB, dstB, sSB)
        issue_idx(g, lo + 3, idxB, sIB)

        def pbody(p, cr):
            i0 = lo + 2 * p
            wait_sc(rowsA, dstA, sSA)
            wait_ones(dstA, sOA)
            wait_idx(idxA, sIA)
            stage1(idxA, dstA, rowsA, sOA, sGA)
            wait_sc(rowsB, dstB, sSB)
            wait_ones(dstB, sOB)
            wait_idx(idxB, sIB)
            stage1(idxB, dstB, rowsB, sOB, sGB)
            wait_g(idxA, rowsA, sGA)
            issue_sc(rowsA, dstA, sSA)
            issue_idx(g, i0 + 2, idxA, sIA)
            wait_g(idxB, rowsB, sGB)
            issue_sc(rowsB, dstB, sSB)
            issue_idx(g, i0 + 3, idxB, sIB)
            return cr
        lax.fori_loop(1, PAIRS, pbody, 0)

        wait_sc(rowsA, dstA, sSA)
        wait_ones(dstA, sOA)
        wait_sc(rowsB, dstB, sSB)
        wait_ones(dstB, sOB)
        wait_idx(idxA, sIA)

        @pl.when(has_rem)
        def _():
            stage1(idxA, dstA, rowsA, sOA, sGA)
            wait_g(idxA, rowsA, sGA)
            issue_sc(rowsA, dstA, sSA)
            wait_sc(rowsA, dstA, sSA)
            wait_ones(dstA, sOA)

        wait_idx(idxB, sIB)
        plsc.subcore_barrier()

        def foldj(j, cr):
            pltpu.async_copy(deg_sp.at[nidx.at[j]], dbuf, sGB).wait()

            def winvb(k, cr2):
                v = dbuf[pl.ds(k * 16, 16)]
                winv[pl.ds(j * C + k * 16, 16)] = (
                    1.0 / jnp.maximum(v, 1.0))
                return cr2
            lax.fori_loop(0, 8, winvb, 0)
            pltpu.async_copy(sum_sp.at[nidx.at[j]], rowsA, sGA).wait()

            def foldg(g16, cr2):
                wg = winv[pl.ds(j * C + g16 * 16, 16)]
                for lane in range(16):
                    w = wg[lane]
                    rl = g16 * 16 + lane
                    for cc in range(8):
                        rowsB[rl, pl.ds(cc * 16, 16)] = (
                            rowsA[rl, pl.ds(cc * 16, 16)] * w)
                return cr2
            lax.fori_loop(0, 8, foldg, 0)
            pltpu.sync_copy(
                rowsB, partial.at[g, c, pl.ds(s * 640 + j * C, C)])
            return cr
        lax.fori_loop(0, 5, foldj, 0)
        plsc.subcore_barrier()
        return carry

    lax.fori_loop(0, 4, graph_body, 0)

    for j in range(5):
        for k in range(8):
            idxA[0, pl.ds(k * 16, 16)] = nidx[j, pl.ds(k * 16, 16)] + coff
        pltpu.async_copy(feat_cat.at[idxA.at[0]], rowsA, sGA).wait()
        pltpu.sync_copy(
            rowsA, out_self.at[c, pl.ds(s * 640 + j * C, C)])


def _sc_aggregate(nodes_pad, edges_all, feat_cat):
    mesh = plsc.VectorSubcoreMesh(
        core_axis_name="c", subcore_axis_name="s",
        num_cores=NC, num_subcores=NS)
    f = pl.kernel(
        _sc_body,
        out_type=[
            jax.ShapeDtypeStruct((NC, BP, F), jnp.float32),
            jax.ShapeDtypeStruct((4, NC, BP, F), jnp.float32),
        ],
        mesh=mesh,
        compiler_params=pltpu.CompilerParams(needs_layout_passes=False),
        scratch_types=[
            pltpu.VMEM_SHARED((N, F), jnp.float32),
            pltpu.VMEM_SHARED((BP,), jnp.float32),
            pltpu.VMEM((C, F), jnp.float32),
            pltpu.VMEM((C, F), jnp.float32),
            pltpu.VMEM((2, C), jnp.int32),
            pltpu.VMEM((2, C), jnp.int32),
            pltpu.VMEM((C,), jnp.int32),
            pltpu.VMEM((C,), jnp.int32),
            pltpu.VMEM((5, C), jnp.int32),
            pltpu.VMEM((640,), jnp.float32),
            pltpu.VMEM((C,), jnp.float32),
            pltpu.VMEM((C,), jnp.float32),
            pltpu.SemaphoreType.DMA,
            pltpu.SemaphoreType.DMA,
            pltpu.SemaphoreType.DMA,
            pltpu.SemaphoreType.DMA,
            pltpu.SemaphoreType.DMA,
            pltpu.SemaphoreType.DMA,
            pltpu.SemaphoreType.DMA,
            pltpu.SemaphoreType.DMA,
        ],
    )
    return f(nodes_pad, edges_all, feat_cat)


RB = 1024


def _mlp_body(selfp, pp, selfn, pn,
              w1ap, w1bp, b1p, w2p, b2p,
              w1an, w1bn, b1n, w2n, b2n, out):
    aggp = pp[0, 0] + pp[1, 0] + pp[2, 0] + pp[3, 0]
    aggn = pn[0, 0] + pn[1, 0] + pn[2, 0] + pn[3, 0]
    hp = jnp.tanh(
        jnp.dot(selfp[0], w1ap[...], preferred_element_type=jnp.float32)
        + jnp.dot(aggp, w1bp[...], preferred_element_type=jnp.float32)
        + b1p[...])
    out[:, 0:F] = (
        jnp.dot(hp, w2p[...], preferred_element_type=jnp.float32) + b2p[...])
    hn = jnp.tanh(
        jnp.dot(selfn[0], w1an[...], preferred_element_type=jnp.float32)
        + jnp.dot(aggn, w1bn[...], preferred_element_type=jnp.float32)
        + b1n[...])
    out[:, F:2 * F] = (
        jnp.dot(hn, w2n[...], preferred_element_type=jnp.float32) + b2n[...])


def _mlp(out_self, partial,
         w1ap, w1bp, b1p, w2p, b2p,
         w1an, w1bn, b1n, w2n, b2n):
    selfp_spec = pl.BlockSpec((1, RB, F), lambda i: (0, i, 0))
    selfn_spec = pl.BlockSpec((1, RB, F), lambda i: (1, i, 0))
    pp_spec = pl.BlockSpec((4, 1, RB, F), lambda i: (0, 0, i, 0))
    pn_spec = pl.BlockSpec((4, 1, RB, F), lambda i: (0, 1, i, 0))
    w_spec = pl.BlockSpec((F, F), lambda i: (0, 0))
    b_spec = pl.BlockSpec((1, F), lambda i: (0, 0))
    return pl.pallas_call(
        _mlp_body,
        grid=(BP // RB,),
        in_specs=[selfp_spec, pp_spec, selfn_spec, pn_spec,
                  w_spec, w_spec, b_spec, w_spec, b_spec,
                  w_spec, w_spec, b_spec, w_spec, b_spec],
        out_specs=pl.BlockSpec((RB, 2 * F), lambda i: (i, 0)),
        out_shape=jax.ShapeDtypeStruct((BP, 2 * F), jnp.float32),
    )(out_self, partial, out_self, partial,
      w1ap, w1bp, b1p, w2p, b2p,
      w1an, w1bn, b1n, w2n, b2n)


def kernel(nodes, edge_index_0, edge_index_1, edge_index_2, edge_index_3,
           feat_pos, feat_neg,
           W1_pos, b1_pos, W2_pos, b2_pos,
           W1_neg, b1_neg, W2_neg, b2_neg):
    nodes_pad = jnp.concatenate(
        [nodes.astype(jnp.int32), jnp.zeros((BP - B,), jnp.int32)])
    edges_all = jnp.stack(
        [edge_index_0.astype(jnp.int32), edge_index_1.astype(jnp.int32),
         edge_index_2.astype(jnp.int32), edge_index_3.astype(jnp.int32)])
    feat_cat = jnp.concatenate([feat_pos, feat_neg], axis=0)
    out_self, partial = _sc_aggregate(nodes_pad, edges_all, feat_cat)
    out = _mlp(
        out_self, partial,
        W1_pos[:F], W1_pos[F:], b1_pos.reshape(1, F), W2_pos,
        b2_pos.reshape(1, F),
        W1_neg[:F], W1_neg[F:], b1_neg.reshape(1, F), W2_neg,
        b2_neg.reshape(1, F))
    return out[:B]


# speedup vs baseline: 1.0443x; 1.0443x over previous
import jax
import jax.numpy as jnp
from jax import lax
from jax.experimental import pallas as pl
from jax.experimental.pallas import tpu as pltpu
from jax.experimental.pallas import tpu_sc as plsc

N = 10000
E = 320000
F = 128
B = 10000
BP = 10240
NC, NS = 2, 16
C = 128
NCHUNK = E // C
PAIRS = 78


def _sc_body(nodes, edges_all, feat_cat,
             out_self, partial,
             sum_sp, deg_sp, rowsA, rowsB, idxA, idxB,
             dstA, dstB, nidx, winv, dbuf, ones_v,
             sIA, sIB, sGA0, sGA1, sGB0, sGB1, sOA, sOB, sSA, sSB):
    c = lax.axis_index("c")
    s = lax.axis_index("s")
    coff = c * N
    zero16 = jnp.zeros((16,), jnp.float32)
    one16 = jnp.ones((16,), jnp.float32)

    for k in range(8):
        ones_v[pl.ds(k * 16, 16)] = one16
    for j in range(5):
        pltpu.sync_copy(nodes.at[pl.ds(s * 640 + j * C, C)], nidx.at[j])

    def issue_idx(g, ch, idxbuf, sem):
        ch = jnp.minimum(ch, NCHUNK - 1)
        pltpu.async_copy(edges_all.at[g, :, pl.ds(ch * C, C)], idxbuf, sem)

    def wait_idx(idxbuf, sem):
        pltpu.make_async_copy(
            edges_all.at[0, :, pl.ds(0, C)], idxbuf, sem).wait()

    def stage1(idxbuf, dstbuf, rowsbuf, semO, semg0, semg1):
        for h in range(2):
            for k in range(4):
                dstbuf[h, pl.ds(k * 16, 16)] = (
                    idxbuf[1, pl.ds(h * 64 + k * 16, 16)])
        for k in range(8):
            idxbuf[0, pl.ds(k * 16, 16)] = idxbuf[0, pl.ds(k * 16, 16)] + coff
        for h in range(2):
            pltpu.async_copy(
                ones_v.at[pl.ds(0, 64)], deg_sp.at[dstbuf.at[h]], semO,
                add=True)
        for h, semg in ((0, semg0), (1, semg1)):
            pltpu.async_copy(
                feat_cat.at[idxbuf.at[0, pl.ds(h * 64, 64)]],
                rowsbuf.at[pl.ds(h * 64, 64)], semg)

    def wait_g(idxbuf, rowsbuf, h, sem):
        pltpu.make_async_copy(
            feat_cat.at[idxbuf.at[0, pl.ds(h * 64, 64)]],
            rowsbuf.at[pl.ds(h * 64, 64)], sem).wait()

    def issue_sc(rowsbuf, dstbuf, h, sem):
        pltpu.async_copy(
            rowsbuf.at[pl.ds(h * 64, 64)], sum_sp.at[dstbuf.at[h]], sem,
            add=True)

    def wait_sc(rowsbuf, dstbuf, sem):
        for h in range(2):
            pltpu.make_async_copy(
                rowsbuf.at[pl.ds(h * 64, 64)], sum_sp.at[dstbuf.at[h]],
                sem).wait()

    def wait_ones(dstbuf, sem):
        for h in range(2):
            pltpu.make_async_copy(
                ones_v.at[pl.ds(0, 64)], deg_sp.at[dstbuf.at[h]], sem).wait()

    lo = s * NCHUNK // NS
    nfull = jnp.minimum(640, N - s * 640) // C
    has_rem = s % 4 == 3

    def graph_body(g, carry):
        def zw(k, cr):
            winv[pl.ds(k * 16, 16)] = zero16
            return cr
        lax.fori_loop(0, 40, zw, 0)
        pltpu.sync_copy(winv.at[pl.ds(0, 640)], deg_sp.at[pl.ds(s * 640, 640)])

        def zr(r, cr):
            for j2 in range(8):
                rowsA[r, pl.ds(j2 * 16, 16)] = zero16
            return cr
        lax.fori_loop(0, C, zr, 0)

        def zacc(i, cr):
            pltpu.sync_copy(rowsA, sum_sp.at[pl.ds(s * 640 + i * C, C)])
            return cr
        lax.fori_loop(0, nfull, zacc, 0)

        @pl.when(s == NS - 1)
        def _():
            pltpu.sync_copy(rowsA.at[pl.ds(0, 16)], sum_sp.at[pl.ds(9984, 16)])
        plsc.subcore_barrier()

        issue_idx(g, lo, idxA, sIA)
        issue_idx(g, lo + 1, idxB, sIB)
        wait_idx(idxA, sIA)
        stage1(idxA, dstA, rowsA, sOA, sGA0, sGA1)
        wait_idx(idxB, sIB)
        stage1(idxB, dstB, rowsB, sOB, sGB0, sGB1)
        wait_g(idxA, rowsA, 0, sGA0)
        issue_sc(rowsA, dstA, 0, sSA)
        wait_g(idxA, rowsA, 1, sGA1)
        issue_sc(rowsA, dstA, 1, sSA)
        issue_idx(g, lo + 2, idxA, sIA)
        wait_g(idxB, rowsB, 0, sGB0)
        issue_sc(rowsB, dstB, 0, sSB)
        wait_g(idxB, rowsB, 1, sGB1)
        issue_sc(rowsB, dstB, 1, sSB)
        issue_idx(g, lo + 3, idxB, sIB)

        def pbody(p, cr):
            i0 = lo + 2 * p
            wait_sc(rowsA, dstA, sSA)
            wait_ones(dstA, sOA)
            wait_idx(idxA, sIA)
            stage1(idxA, dstA, rowsA, sOA, sGA0, sGA1)
            wait_sc(rowsB, dstB, sSB)
            wait_ones(dstB, sOB)
            wait_idx(idxB, sIB)
            stage1(idxB, dstB, rowsB, sOB, sGB0, sGB1)
            wait_g(idxA, rowsA, 0, sGA0)
            issue_sc(rowsA, dstA, 0, sSA)
            wait_g(idxA, rowsA, 1, sGA1)
            issue_sc(rowsA, dstA, 1, sSA)
            issue_idx(g, i0 + 2, idxA, sIA)
            wait_g(idxB, rowsB, 0, sGB0)
            issue_sc(rowsB, dstB, 0, sSB)
            wait_g(idxB, rowsB, 1, sGB1)
            issue_sc(rowsB, dstB, 1, sSB)
            issue_idx(g, i0 + 3, idxB, sIB)
            return cr
        lax.fori_loop(1, PAIRS, pbody, 0)

        wait_sc(rowsA, dstA, sSA)
        wait_ones(dstA, sOA)
        wait_sc(rowsB, dstB, sSB)
        wait_ones(dstB, sOB)
        wait_idx(idxA, sIA)

        @pl.when(has_rem)
        def _():
            stage1(idxA, dstA, rowsA, sOA, sGA0, sGA1)
            wait_g(idxA, rowsA, 0, sGA0)
            issue_sc(rowsA, dstA, 0, sSA)
            wait_g(idxA, rowsA, 1, sGA1)
            issue_sc(rowsA, dstA, 1, sSA)
            wait_sc(rowsA, dstA, sSA)
            wait_ones(dstA, sOA)

        wait_idx(idxB, sIB)
        plsc.subcore_barrier()

        def foldj(j, cr):
            pltpu.async_copy(deg_sp.at[nidx.at[j]], dbuf, sGB0).wait()

            def winvb(k, cr2):
                v = dbuf[pl.ds(k * 16, 16)]
                winv[pl.ds(j * C + k * 16, 16)] = (
                    1.0 / jnp.maximum(v, 1.0))
                return cr2
            lax.fori_loop(0, 8, winvb, 0)
            pltpu.async_copy(sum_sp.at[nidx.at[j]], rowsA, sGA0).wait()

            def foldg(g16, cr2):
                wg = winv[pl.ds(j * C + g16 * 16, 16)]
                for lane in range(16):
                    w = wg[lane]
                    rl = g16 * 16 + lane
                    for cc in range(8):
                        rowsB[rl, pl.ds(cc * 16, 16)] = (
                            rowsA[rl, pl.ds(cc * 16, 16)] * w)
                return cr2
            lax.fori_loop(0, 8, foldg, 0)
            pltpu.sync_copy(
                rowsB, partial.at[g, c, pl.ds(s * 640 + j * C, C)])
            return cr
        lax.fori_loop(0, 5, foldj, 0)
        plsc.subcore_barrier()
        return carry

    lax.fori_loop(0, 4, graph_body, 0)

    for j in range(5):
        for k in range(8):
            idxA[0, pl.ds(k * 16, 16)] = nidx[j, pl.ds(k * 16, 16)] + coff
        pltpu.async_copy(feat_cat.at[idxA.at[0]], rowsA, sGA0).wait()
        pltpu.sync_copy(
            rowsA, out_self.at[c, pl.ds(s * 640 + j * C, C)])


def _sc_aggregate(nodes_pad, edges_all, feat_cat):
    mesh = plsc.VectorSubcoreMesh(
        core_axis_name="c", subcore_axis_name="s",
        num_cores=NC, num_subcores=NS)
    f = pl.kernel(
        _sc_body,
        out_type=[
            jax.ShapeDtypeStruct((NC, BP, F), jnp.float32),
            jax.ShapeDtypeStruct((4, NC, BP, F), jnp.float32),
        ],
        mesh=mesh,
        compiler_params=pltpu.CompilerParams(needs_layout_passes=False),
        scratch_types=[
            pltpu.VMEM_SHARED((N, F), jnp.float32),
            pltpu.VMEM_SHARED((BP,), jnp.float32),
            pltpu.VMEM((C, F), jnp.float32),
            pltpu.VMEM((C, F), jnp.float32),
            pltpu.VMEM((2, C), jnp.int32),
            pltpu.VMEM((2, C), jnp.int32),
            pltpu.VMEM((2, 64), jnp.int32),
            pltpu.VMEM((2, 64), jnp.int32),
            pltpu.VMEM((5, C), jnp.int32),
            pltpu.VMEM((640,), jnp.float32),
            pltpu.VMEM((C,), jnp.float32),
            pltpu.VMEM((C,), jnp.float32),
            pltpu.SemaphoreType.DMA,
            pltpu.SemaphoreType.DMA,
            pltpu.SemaphoreType.DMA,
            pltpu.SemaphoreType.DMA,
            pltpu.SemaphoreType.DMA,
            pltpu.SemaphoreType.DMA,
            pltpu.SemaphoreType.DMA,
            pltpu.SemaphoreType.DMA,
            pltpu.SemaphoreType.DMA,
            pltpu.SemaphoreType.DMA,
        ],
    )
    return f(nodes_pad, edges_all, feat_cat)


RB = 1024


def _mlp_body(selfp, pp, selfn, pn,
              w1ap, w1bp, b1p, w2p, b2p,
              w1an, w1bn, b1n, w2n, b2n, out):
    aggp = pp[0, 0] + pp[1, 0] + pp[2, 0] + pp[3, 0]
    aggn = pn[0, 0] + pn[1, 0] + pn[2, 0] + pn[3, 0]
    hp = jnp.tanh(
        jnp.dot(selfp[0], w1ap[...], preferred_element_type=jnp.float32)
        + jnp.dot(aggp, w1bp[...], preferred_element_type=jnp.float32)
        + b1p[...])
    out[:, 0:F] = (
        jnp.dot(hp, w2p[...], preferred_element_type=jnp.float32) + b2p[...])
    hn = jnp.tanh(
        jnp.dot(selfn[0], w1an[...], preferred_element_type=jnp.float32)
        + jnp.dot(aggn, w1bn[...], preferred_element_type=jnp.float32)
        + b1n[...])
    out[:, F:2 * F] = (
        jnp.dot(hn, w2n[...], preferred_element_type=jnp.float32) + b2n[...])


def _mlp(out_self, partial,
         w1ap, w1bp, b1p, w2p, b2p,
         w1an, w1bn, b1n, w2n, b2n):
    selfp_spec = pl.BlockSpec((1, RB, F), lambda i: (0, i, 0))
    selfn_spec = pl.BlockSpec((1, RB, F), lambda i: (1, i, 0))
    pp_spec = pl.BlockSpec((4, 1, RB, F), lambda i: (0, 0, i, 0))
    pn_spec = pl.BlockSpec((4, 1, RB, F), lambda i: (0, 1, i, 0))
    w_spec = pl.BlockSpec((F, F), lambda i: (0, 0))
    b_spec = pl.BlockSpec((1, F), lambda i: (0, 0))
    return pl.pallas_call(
        _mlp_body,
        grid=(BP // RB,),
        in_specs=[selfp_spec, pp_spec, selfn_spec, pn_spec,
                  w_spec, w_spec, b_spec, w_spec, b_spec,
                  w_spec, w_spec, b_spec, w_spec, b_spec],
        out_specs=pl.BlockSpec((RB, 2 * F), lambda i: (i, 0)),
        out_shape=jax.ShapeDtypeStruct((BP, 2 * F), jnp.float32),
    )(out_self, partial, out_self, partial,
      w1ap, w1bp, b1p, w2p, b2p,
      w1an, w1bn, b1n, w2n, b2n)


def kernel(nodes, edge_index_0, edge_index_1, edge_index_2, edge_index_3,
           feat_pos, feat_neg,
           W1_pos, b1_pos, W2_pos, b2_pos,
           W1_neg, b1_neg, W2_neg, b2_neg):
    nodes_pad = jnp.concatenate(
        [nodes.astype(jnp.int32), jnp.zeros((BP - B,), jnp.int32)])
    edges_all = jnp.stack(
        [edge_index_0.astype(jnp.int32), edge_index_1.astype(jnp.int32),
         edge_index_2.astype(jnp.int32), edge_index_3.astype(jnp.int32)])
    feat_cat = jnp.concatenate([feat_pos, feat_neg], axis=0)
    out_self, partial = _sc_aggregate(nodes_pad, edges_all, feat_cat)
    out = _mlp(
        out_self, partial,
        W1_pos[:F], W1_pos[F:], b1_pos.reshape(1, F), W2_pos,
        b2_pos.reshape(1, F),
        W1_neg[:F], W1_neg[F:], b1_neg.reshape(1, F), W2_neg,
        b2_neg.reshape(1, F))
    return out[:B]
